# baseline (speedup 1.0000x reference)
import functools

import jax
import jax.numpy as jnp
from jax.experimental import pallas as pl
from jax.experimental.pallas import tpu as pltpu

_E = 160000
_T = 640000
_H = 256
_NR = 6
_NSR = 42
_IE = 64

_BLK_A = 2000
_BLK_B = 8000
_BLK_D = 2000


def _swish(x):
    return x * jax.nn.sigmoid(x)


def _mm(a, b):
    return jax.lax.dot_general(a, b, (((1,), (0,)), ((), ())),
                               preferred_element_type=jnp.float32)


def _stage_a_body(x1_ref, rbf0_ref, Wji_ref, bji_ref, Wkj_ref, bkj_ref,
                  Wr1_ref, Wr2_ref, Wdown_ref, xji_ref, xd_ref):
    x1 = x1_ref[...]
    xji_ref[...] = _swish(_mm(x1, Wji_ref[...]) + bji_ref[...])
    rbf = _mm(_mm(rbf0_ref[...], Wr1_ref[...]), Wr2_ref[...])
    xkj = _swish(_mm(x1, Wkj_ref[...]) + bkj_ref[...]) * rbf
    xd_ref[...] = _swish(_mm(xkj, Wdown_ref[...]))


def _stage_b_body(sbf_ref, Ws1_ref, Ws2_ref, sbf2_ref):
    sbf2_ref[...] = _mm(_mm(sbf_ref[...], Ws1_ref[...]), Ws2_ref[...])


def _stage_d_body(agg_ref, xji_ref, x1_ref, rbf0_ref, Wup_ref,
                  Wb01_ref, bb01_ref, Wb02_ref, bb02_ref,
                  Wlin_ref, blin_ref,
                  Wa01_ref, ba01_ref, Wa02_ref, ba02_ref,
                  Wa11_ref, ba11_ref, Wa12_ref, ba12_ref,
                  Wrbf_ref, e1_ref, e2_ref):
    xu = _swish(_mm(agg_ref[...], Wup_ref[...]))
    e1 = xji_ref[...] + xu
    e1 = e1 + _swish(_mm(_swish(_mm(e1, Wb01_ref[...]) + bb01_ref[...]),
                         Wb02_ref[...]) + bb02_ref[...])
    e1 = _swish(_mm(e1, Wlin_ref[...]) + blin_ref[...]) + x1_ref[...]
    e1 = e1 + _swish(_mm(_swish(_mm(e1, Wa01_ref[...]) + ba01_ref[...]),
                         Wa02_ref[...]) + ba02_ref[...])
    e1 = e1 + _swish(_mm(_swish(_mm(e1, Wa11_ref[...]) + ba11_ref[...]),
                         Wa12_ref[...]) + ba12_ref[...])
    e1_ref[...] = e1
    e2_ref[...] = _mm(rbf0_ref[...], Wrbf_ref[...]) * e1


def _full(shape):
    return pl.BlockSpec(shape, lambda i: tuple(0 for _ in shape))


def _rows(blk, width):
    return pl.BlockSpec((blk, width), lambda i: (i, 0))


def _stage_a(x1, rbf0, W_ji, b_ji, W_kj, b_kj, W_rbf1, W_rbf2, W_down):
    grid = (_E // _BLK_A,)
    return pl.pallas_call(
        _stage_a_body,
        grid=grid,
        in_specs=[
            _rows(_BLK_A, _H), _rows(_BLK_A, _NR),
            _full((_H, _H)), _full((1, _H)), _full((_H, _H)), _full((1, _H)),
            _full((_NR, 8)), _full((8, _H)), _full((_H, _IE)),
        ],
        out_specs=[_rows(_BLK_A, _H), _rows(_BLK_A, _IE)],
        out_shape=[
            jax.ShapeDtypeStruct((_E, _H), jnp.float32),
            jax.ShapeDtypeStruct((_E, _IE), jnp.float32),
        ],
        compiler_params=pltpu.CompilerParams(
            dimension_semantics=("parallel",)),
    )(x1, rbf0, W_ji, b_ji, W_kj, b_kj, W_rbf1, W_rbf2, W_down)


def _stage_b(sbf, W_sbf1, W_sbf2):
    grid = (_T // _BLK_B,)
    return pl.pallas_call(
        _stage_b_body,
        grid=grid,
        in_specs=[_rows(_BLK_B, _NSR), _full((_NSR, 8)), _full((8, _IE))],
        out_specs=_rows(_BLK_B, _IE),
        out_shape=jax.ShapeDtypeStruct((_T, _IE), jnp.float32),
        compiler_params=pltpu.CompilerParams(
            dimension_semantics=("parallel",)),
    )(sbf, W_sbf1, W_sbf2)


def _stage_d(agg, xji, x1, rbf0, W_up, Wb0_1, bb0_1, Wb0_2, bb0_2,
             W_lin, b_lin, Wa0_1, ba0_1, Wa0_2, ba0_2,
             Wa1_1, ba1_1, Wa1_2, ba1_2, W_rbf):
    grid = (_E // _BLK_D,)
    return pl.pallas_call(
        _stage_d_body,
        grid=grid,
        in_specs=[
            _rows(_BLK_D, _IE), _rows(_BLK_D, _H), _rows(_BLK_D, _H),
            _rows(_BLK_D, _NR),
            _full((_IE, _H)),
            _full((_H, _H)), _full((1, _H)), _full((_H, _H)), _full((1, _H)),
            _full((_H, _H)), _full((1, _H)),
            _full((_H, _H)), _full((1, _H)), _full((_H, _H)), _full((1, _H)),
            _full((_H, _H)), _full((1, _H)), _full((_H, _H)), _full((1, _H)),
            _full((_NR, _H)),
        ],
        out_specs=[_rows(_BLK_D, _H), _rows(_BLK_D, _H)],
        out_shape=[
            jax.ShapeDtypeStruct((_E, _H), jnp.float32),
            jax.ShapeDtypeStruct((_E, _H), jnp.float32),
        ],
        compiler_params=pltpu.CompilerParams(
            dimension_semantics=("parallel",)),
    )(agg, xji, x1, rbf0, W_up, Wb0_1, bb0_1, Wb0_2, bb0_2, W_lin, b_lin,
      Wa0_1, ba0_1, Wa0_2, ba0_2, Wa1_1, ba1_1, Wa1_2, ba1_2, W_rbf)


def kernel(x1, x2, rbf0, sbf, W_rbf1, W_rbf2, W_sbf1, W_sbf2, W_rbf, W_kj,
           b_kj, W_ji, b_ji, W_down, W_up, Wb0_1, bb0_1, Wb0_2, bb0_2,
           W_lin, b_lin, Wa0_1, ba0_1, Wa0_2, ba0_2, Wa1_1, ba1_1, Wa1_2,
           ba1_2, idx_kj, idx_ji):
    del x2
    r2 = lambda b: b.reshape(1, _H)
    xji, xd = _stage_a(x1, rbf0, W_ji, r2(b_ji), W_kj, r2(b_kj),
                       W_rbf1, W_rbf2, W_down)
    sbf2 = _stage_b(sbf, W_sbf1, W_sbf2)
    msg = jnp.take(xd, idx_kj, axis=0) * sbf2
    agg = jax.ops.segment_sum(msg, idx_ji, num_segments=_E)
    e1, e2 = _stage_d(agg, xji, x1, rbf0, W_up, Wb0_1, r2(bb0_1), Wb0_2,
                      r2(bb0_2), W_lin, r2(b_lin), Wa0_1, r2(ba0_1), Wa0_2,
                      r2(ba0_2), Wa1_1, r2(ba1_1), Wa1_2, r2(ba1_2), W_rbf)
    return (e1, e2)


# speedup vs baseline: 3.4120x; 3.4120x over previous
import functools

import jax
import jax.numpy as jnp
from jax import lax
from jax.experimental import pallas as pl
from jax.experimental.pallas import tpu as pltpu
from jax.experimental.pallas import tpu_sc as plsc

_E = 160000
_T = 640000
_H = 256
_NR = 6
_NSR = 42
_IE = 64

_BLK_A = 2000
_BLK_B = 8000
_BLK_D = 2000

_NF = 16
_NSL = 4
_GV = 64
_K = 5
_G = 2048
_NSUB = 16
_UNROLL = 8

_R = _T // _GV
_CPT = _R // _NSUB
_NCH = _CPT // _K
_C = _E // 2
_ZPT = (_C + _G) // _NSUB
_DPT = _C // _NSUB


def _swish(x):
    return x * jax.nn.sigmoid(x)


def _mm(a, b):
    return jax.lax.dot_general(a, b, (((1,), (0,)), ((), ())),
                               preferred_element_type=jnp.float32)


def _stage_a_body(x1_ref, rbf0_ref, Wji_ref, bji_ref, Wkj_ref, bkj_ref,
                  Wr1_ref, Wr2_ref, Wdown_ref, xji_ref, t0_ref, t1_ref,
                  t2_ref, t3_ref):
    x1 = x1_ref[...]
    xji_ref[...] = _swish(_mm(x1, Wji_ref[...]) + bji_ref[...])
    rbf = _mm(_mm(rbf0_ref[...], Wr1_ref[...]), Wr2_ref[...])
    xkj = _swish(_mm(x1, Wkj_ref[...]) + bkj_ref[...]) * rbf
    xd = _swish(_mm(xkj, Wdown_ref[...]))
    for r, ref in enumerate((t0_ref, t1_ref, t2_ref, t3_ref)):
        ref[...] = xd[:, r * _NF:(r + 1) * _NF]


def _stage_a(x1, rbf0, W_ji, b_ji, W_kj, b_kj, W_rbf1, W_rbf2, W_down):
    grid = (_E // _BLK_A,)
    return pl.pallas_call(
        _stage_a_body,
        grid=grid,
        in_specs=[
            _rows(_BLK_A, _H), _rows(_BLK_A, _NR),
            _full((_H, _H)), _full((1, _H)), _full((_H, _H)), _full((1, _H)),
            _full((_NR, 8)), _full((8, _H)), _full((_H, _IE)),
        ],
        out_specs=[_rows(_BLK_A, _H)] + [_rows(_BLK_A, _NF)] * _NSL,
        out_shape=[jax.ShapeDtypeStruct((_E, _H), jnp.float32)]
        + [jax.ShapeDtypeStruct((_E, _NF), jnp.float32)] * _NSL,
        compiler_params=pltpu.CompilerParams(
            dimension_semantics=("parallel",)),
    )(x1, rbf0, W_ji, b_ji, W_kj, b_kj, W_rbf1, W_rbf2, W_down)


def _stage_b_body(sbf_ref, Ws1_ref, Ws2_ref, p0_ref, p1_ref, p2_ref, p3_ref):
    s2 = _mm(_mm(sbf_ref[...], Ws1_ref[...]), Ws2_ref[...])
    for r, ref in enumerate((p0_ref, p1_ref, p2_ref, p3_ref)):
        ref[...] = s2[:, r * _NF:(r + 1) * _NF]


def _stage_b(sbf, W_sbf1, W_sbf2):
    grid = (_T // _BLK_B,)
    return pl.pallas_call(
        _stage_b_body,
        grid=grid,
        in_specs=[_rows(_BLK_B, _NSR), _full((_NSR, 8)), _full((8, _IE))],
        out_specs=[_rows(_BLK_B, _NF)] * _NSL,
        out_shape=[jax.ShapeDtypeStruct((_T, _NF), jnp.float32)] * _NSL,
        compiler_params=pltpu.CompilerParams(
            dimension_semantics=("parallel",)),
    )(sbf, W_sbf1, W_sbf2)


_sc_mesh = plsc.VectorSubcoreMesh(core_axis_name="c", subcore_axis_name="s")


@pl.kernel(
    out_type=[jax.ShapeDtypeStruct((_E, _NF), jnp.float32)] * _NSL,
    mesh=_sc_mesh,
    scratch_types=[
        pltpu.VMEM((_K, _GV), jnp.int32),
        pltpu.VMEM((_K, _GV), jnp.int32),
        pltpu.VMEM((_K, _GV), jnp.int32),
        pltpu.VMEM((_K * _GV, _NF), jnp.float32),
        pltpu.VMEM((_K * _GV, _NF), jnp.float32),
        pltpu.VMEM_SHARED((_C + _G, _NF), jnp.float32),
        pltpu.SemaphoreType.DMA,
        pltpu.SemaphoreType.DMA,
    ],
    compiler_params=pltpu.CompilerParams(use_tc_tiling_on_sc=False),
)
def _sc_sparse(t0, t1, t2, t3, p0, p1, p2, p3, kj2, ji2, zeros_hbm,
               o0, o1, o2, o3,
               kj_b, ji_b, dest_b, rows_b, sb_b, acc, gsem, ssem):
    c = lax.axis_index("c")
    w = lax.axis_index("s")
    base = (c * _C).astype(jnp.int32)
    basev = jnp.full((16,), 0, jnp.int32) + base
    row0 = w * _CPT
    tables = (t0, t1, t2, t3)
    packs = (p0, p1, p2, p3)
    outs = (o0, o1, o2, o3)

    for r in range(_NSL):
        tbl = tables[r]
        pck = packs[r]
        pltpu.sync_copy(zeros_hbm.at[pl.ds(w * _ZPT, _ZPT)],
                        acc.at[pl.ds(w * _ZPT, _ZPT)])
        plsc.subcore_barrier()

        def chunk_body(ci, _, tbl=tbl, pck=pck):
            r0 = row0 + ci * _K
            pltpu.sync_copy(kj2.at[pl.ds(r0, _K)], kj_b)
            pltpu.sync_copy(ji2.at[pl.ds(r0, _K)], ji_b)
            sb_copy = pltpu.async_copy(
                pck.at[pl.ds(r0 * _GV, _K * _GV)], sb_b, gsem)
            gathers = [
                pltpu.async_copy(tbl.at[kj_b.at[kk]],
                                 rows_b.at[pl.ds(kk * _GV, _GV)], gsem)
                for kk in range(_K)
            ]
            for k in range(_K):
                for j in range(_GV // 16):
                    sl = (k, pl.ds(j * 16, 16))
                    v = ji_b[sl]
                    d = v - basev
                    ok = (v >= basev) & (d < _C)
                    garb = (v & (_G - 1)) + _C
                    dest_b[sl] = jnp.where(ok, d, garb)
            sb_copy.wait()
            for g in gathers:
                g.wait()

            def mul_body(i, _):
                for u in range(_UNROLL):
                    rr = i * _UNROLL + u
                    rows_b[rr, :] = rows_b[rr, :] * sb_b[rr, :]
                return 0

            lax.fori_loop(0, _K * _GV // _UNROLL, mul_body, 0)
            scatters = [
                pltpu.async_copy(rows_b.at[pl.ds(kk * _GV, _GV)],
                                 acc.at[dest_b.at[kk]], ssem, add=True)
                for kk in range(_K)
            ]
            for s in scatters:
                s.wait()
            return 0

        lax.fori_loop(0, _NCH, chunk_body, 0)
        plsc.subcore_barrier()
        pltpu.sync_copy(acc.at[pl.ds(w * _DPT, _DPT)],
                        outs[r].at[pl.ds(c * _C + w * _DPT, _DPT)])
        plsc.subcore_barrier()


def _stage_d_body(a0_ref, a1_ref, a2_ref, a3_ref, xji_ref, x1_ref, rbf0_ref,
                  Wup_ref,
                  Wb01_ref, bb01_ref, Wb02_ref, bb02_ref,
                  Wlin_ref, blin_ref,
                  Wa01_ref, ba01_ref, Wa02_ref, ba02_ref,
                  Wa11_ref, ba11_ref, Wa12_ref, ba12_ref,
                  Wrbf_ref, e1_ref, e2_ref):
    pre = _mm(a0_ref[...], Wup_ref[0 * _NF:1 * _NF, :])
    pre += _mm(a1_ref[...], Wup_ref[1 * _NF:2 * _NF, :])
    pre += _mm(a2_ref[...], Wup_ref[2 * _NF:3 * _NF, :])
    pre += _mm(a3_ref[...], Wup_ref[3 * _NF:4 * _NF, :])
    xu = _swish(pre)
    e1 = xji_ref[...] + xu
    e1 = e1 + _swish(_mm(_swish(_mm(e1, Wb01_ref[...]) + bb01_ref[...]),
                         Wb02_ref[...]) + bb02_ref[...])
    e1 = _swish(_mm(e1, Wlin_ref[...]) + blin_ref[...]) + x1_ref[...]
    e1 = e1 + _swish(_mm(_swish(_mm(e1, Wa01_ref[...]) + ba01_ref[...]),
                         Wa02_ref[...]) + ba02_ref[...])
    e1 = e1 + _swish(_mm(_swish(_mm(e1, Wa11_ref[...]) + ba11_ref[...]),
                         Wa12_ref[...]) + ba12_ref[...])
    e1_ref[...] = e1
    e2_ref[...] = _mm(rbf0_ref[...], Wrbf_ref[...]) * e1


def _full(shape):
    return pl.BlockSpec(shape, lambda i: tuple(0 for _ in shape))


def _rows(blk, width):
    return pl.BlockSpec((blk, width), lambda i: (i, 0))


def _stage_d(aggs, xji, x1, rbf0, W_up, Wb0_1, bb0_1, Wb0_2, bb0_2,
             W_lin, b_lin, Wa0_1, ba0_1, Wa0_2, ba0_2,
             Wa1_1, ba1_1, Wa1_2, ba1_2, W_rbf):
    grid = (_E // _BLK_D,)
    return pl.pallas_call(
        _stage_d_body,
        grid=grid,
        in_specs=[
            _rows(_BLK_D, _NF), _rows(_BLK_D, _NF), _rows(_BLK_D, _NF),
            _rows(_BLK_D, _NF),
            _rows(_BLK_D, _H), _rows(_BLK_D, _H), _rows(_BLK_D, _NR),
            _full((_IE, _H)),
            _full((_H, _H)), _full((1, _H)), _full((_H, _H)), _full((1, _H)),
            _full((_H, _H)), _full((1, _H)),
            _full((_H, _H)), _full((1, _H)), _full((_H, _H)), _full((1, _H)),
            _full((_H, _H)), _full((1, _H)), _full((_H, _H)), _full((1, _H)),
            _full((_NR, _H)),
        ],
        out_specs=[_rows(_BLK_D, _H), _rows(_BLK_D, _H)],
        out_shape=[
            jax.ShapeDtypeStruct((_E, _H), jnp.float32),
            jax.ShapeDtypeStruct((_E, _H), jnp.float32),
        ],
        compiler_params=pltpu.CompilerParams(
            dimension_semantics=("parallel",)),
    )(*aggs, xji, x1, rbf0, W_up, Wb0_1, bb0_1, Wb0_2, bb0_2, W_lin, b_lin,
      Wa0_1, ba0_1, Wa0_2, ba0_2, Wa1_1, ba1_1, Wa1_2, ba1_2, W_rbf)


def kernel(x1, x2, rbf0, sbf, W_rbf1, W_rbf2, W_sbf1, W_sbf2, W_rbf, W_kj,
           b_kj, W_ji, b_ji, W_down, W_up, Wb0_1, bb0_1, Wb0_2, bb0_2,
           W_lin, b_lin, Wa0_1, ba0_1, Wa0_2, ba0_2, Wa1_1, ba1_1, Wa1_2,
           ba1_2, idx_kj, idx_ji):
    del x2
    r2 = lambda b: b.reshape(1, _H)
    xji, t0, t1, t2, t3 = _stage_a(x1, rbf0, W_ji, r2(b_ji), W_kj, r2(b_kj),
                                   W_rbf1, W_rbf2, W_down)
    p0, p1, p2, p3 = _stage_b(sbf, W_sbf1, W_sbf2)
    kj2 = idx_kj.astype(jnp.int32).reshape(_R, _GV)
    ji2 = idx_ji.astype(jnp.int32).reshape(_R, _GV)
    zeros = jnp.zeros((_C + _G, _NF), jnp.float32)
    aggs = _sc_sparse(t0, t1, t2, t3, p0, p1, p2, p3, kj2, ji2, zeros)
    e1, e2 = _stage_d(aggs, xji, x1, rbf0, W_up, Wb0_1, r2(bb0_1), Wb0_2,
                      r2(bb0_2), W_lin, r2(b_lin), Wa0_1, r2(ba0_1), Wa0_2,
                      r2(ba0_2), Wa1_1, r2(ba1_1), Wa1_2, r2(ba1_2), W_rbf)
    return (e1, e2)


# speedup vs baseline: 4.3445x; 1.2733x over previous
import functools

import jax
import jax.numpy as jnp
from jax import lax
from jax.experimental import pallas as pl
from jax.experimental.pallas import tpu as pltpu
from jax.experimental.pallas import tpu_sc as plsc

_E = 160000
_T = 640000
_H = 256
_NR = 6
_NSR = 42
_IE = 64

_BLK_A = 2000
_BLK_B = 8000
_BLK_D = 2000

_NF = 16
_NSL = 4
_GV = 64
_K = 5
_G = 2048
_NSUB = 16
_UNROLL = 8

_R = _T // _GV
_CPT = _R // _NSUB
_NCH = _CPT // _K
_C = _E // 2
_ZPT = (_C + _G) // _NSUB
_DPT = _C // _NSUB


def _swish(x):
    return x * jax.nn.sigmoid(x)


def _mm(a, b):
    return jax.lax.dot_general(a, b, (((1,), (0,)), ((), ())),
                               preferred_element_type=jnp.float32)


def _stage_a_body(x1_ref, rbf0_ref, Wji_ref, bji_ref, Wkj_ref, bkj_ref,
                  Wr1_ref, Wr2_ref, Wdown_ref, xji_ref, xd_ref):
    x1 = x1_ref[...]
    xji_ref[...] = _swish(_mm(x1, Wji_ref[...]) + bji_ref[...])
    rbf = _mm(_mm(rbf0_ref[...], Wr1_ref[...]), Wr2_ref[...])
    xkj = _swish(_mm(x1, Wkj_ref[...]) + bkj_ref[...]) * rbf
    xd_ref[...] = _swish(_mm(xkj, Wdown_ref[...]))


def _stage_a(x1, rbf0, W_ji, b_ji, W_kj, b_kj, W_rbf1, W_rbf2, W_down):
    grid = (_E // _BLK_A,)
    return pl.pallas_call(
        _stage_a_body,
        grid=grid,
        in_specs=[
            _rows(_BLK_A, _H), _rows(_BLK_A, _NR),
            _full((_H, _H)), _full((1, _H)), _full((_H, _H)), _full((1, _H)),
            _full((_NR, 8)), _full((8, _H)), _full((_H, _IE)),
        ],
        out_specs=[_rows(_BLK_A, _H), _rows(_BLK_A, _IE)],
        out_shape=[jax.ShapeDtypeStruct((_E, _H), jnp.float32),
                   jax.ShapeDtypeStruct((_E, _IE), jnp.float32)],
        compiler_params=pltpu.CompilerParams(
            dimension_semantics=("parallel",)),
    )(x1, rbf0, W_ji, b_ji, W_kj, b_kj, W_rbf1, W_rbf2, W_down)


def _stage_b_body(sbf_ref, Ws1_ref, Ws2_ref, sbf2_ref):
    sbf2_ref[...] = _mm(_mm(sbf_ref[...], Ws1_ref[...]), Ws2_ref[...])


def _stage_b(sbf, W_sbf1, W_sbf2):
    grid = (_T // _BLK_B,)
    return pl.pallas_call(
        _stage_b_body,
        grid=grid,
        in_specs=[_rows(_BLK_B, _NSR), _full((_NSR, 8)), _full((8, _IE))],
        out_specs=_rows(_BLK_B, _IE),
        out_shape=jax.ShapeDtypeStruct((_T, _IE), jnp.float32),
        compiler_params=pltpu.CompilerParams(
            dimension_semantics=("parallel",)),
    )(sbf, W_sbf1, W_sbf2)


_sc_mesh = plsc.VectorSubcoreMesh(core_axis_name="c", subcore_axis_name="s")


@pl.kernel(
    out_type=jax.ShapeDtypeStruct((_E, _IE), jnp.float32),
    mesh=_sc_mesh,
    scratch_types=[
        pltpu.VMEM((_K, _GV), jnp.int32),
        pltpu.VMEM((_K, _GV), jnp.int32),
        pltpu.VMEM((_K, _GV), jnp.int32),
        pltpu.VMEM((_K * _GV, _NF), jnp.float32),
        pltpu.VMEM((_K * _GV, _NF), jnp.float32),
        pltpu.VMEM_SHARED((_C + _G, _NF), jnp.float32),
        pltpu.SemaphoreType.DMA,
        pltpu.SemaphoreType.DMA,
    ],
    compiler_params=pltpu.CompilerParams(use_tc_tiling_on_sc=False),
)
def _sc_sparse(xdT, sbf2, kj2, ji2, zeros_hbm, out,
               kj_b, ji_b, dest_b, rows_b, sb_b, acc, gsem, ssem):
    c = lax.axis_index("c")
    w = lax.axis_index("s")
    base = (c * _C).astype(jnp.int32)
    basev = jnp.full((16,), 0, jnp.int32) + base
    row0 = w * _CPT

    for r in range(_NSL):
        f0 = r * _NF
        offv = jnp.full((16,), r * _E, jnp.int32)
        pltpu.sync_copy(zeros_hbm.at[pl.ds(w * _ZPT, _ZPT)],
                        acc.at[pl.ds(w * _ZPT, _ZPT)])
        plsc.subcore_barrier()

        def chunk_body(ci, _, f0=f0, offv=offv):
            r0 = row0 + ci * _K
            pltpu.sync_copy(kj2.at[pl.ds(r0, _K)], kj_b)
            pltpu.sync_copy(ji2.at[pl.ds(r0, _K)], ji_b)
            sb_copy = pltpu.async_copy(
                sbf2.at[pl.ds(r0 * _GV, _K * _GV), pl.ds(f0, _NF)],
                sb_b, gsem)
            for k in range(_K):
                for j in range(_GV // 16):
                    sl = (k, pl.ds(j * 16, 16))
                    kj_b[sl] = kj_b[sl] + offv
                    v = ji_b[sl]
                    d = v - basev
                    ok = (v >= basev) & (d < _C)
                    garb = (v & (_G - 1)) + _C
                    dest_b[sl] = jnp.where(ok, d, garb)
            gathers = [
                pltpu.async_copy(xdT.at[kj_b.at[kk]],
                                 rows_b.at[pl.ds(kk * _GV, _GV)], gsem)
                for kk in range(_K)
            ]
            sb_copy.wait()
            for g in gathers:
                g.wait()

            def mul_body(i, _):
                for u in range(_UNROLL):
                    rr = i * _UNROLL + u
                    rows_b[rr, :] = rows_b[rr, :] * sb_b[rr, :]
                return 0

            lax.fori_loop(0, _K * _GV // _UNROLL, mul_body, 0)
            scatters = [
                pltpu.async_copy(rows_b.at[pl.ds(kk * _GV, _GV)],
                                 acc.at[dest_b.at[kk]], ssem, add=True)
                for kk in range(_K)
            ]
            for s in scatters:
                s.wait()
            return 0

        lax.fori_loop(0, _NCH, chunk_body, 0)
        plsc.subcore_barrier()
        pltpu.sync_copy(acc.at[pl.ds(w * _DPT, _DPT)],
                        out.at[pl.ds(c * _C + w * _DPT, _DPT),
                               pl.ds(f0, _NF)])
        plsc.subcore_barrier()


def _stage_d_body(agg_ref, xji_ref, x1_ref, rbf0_ref,
                  Wup_ref,
                  Wb01_ref, bb01_ref, Wb02_ref, bb02_ref,
                  Wlin_ref, blin_ref,
                  Wa01_ref, ba01_ref, Wa02_ref, ba02_ref,
                  Wa11_ref, ba11_ref, Wa12_ref, ba12_ref,
                  Wrbf_ref, e1_ref, e2_ref):
    xu = _swish(_mm(agg_ref[...], Wup_ref[...]))
    e1 = xji_ref[...] + xu
    e1 = e1 + _swish(_mm(_swish(_mm(e1, Wb01_ref[...]) + bb01_ref[...]),
                         Wb02_ref[...]) + bb02_ref[...])
    e1 = _swish(_mm(e1, Wlin_ref[...]) + blin_ref[...]) + x1_ref[...]
    e1 = e1 + _swish(_mm(_swish(_mm(e1, Wa01_ref[...]) + ba01_ref[...]),
                         Wa02_ref[...]) + ba02_ref[...])
    e1 = e1 + _swish(_mm(_swish(_mm(e1, Wa11_ref[...]) + ba11_ref[...]),
                         Wa12_ref[...]) + ba12_ref[...])
    e1_ref[...] = e1
    e2_ref[...] = _mm(rbf0_ref[...], Wrbf_ref[...]) * e1


def _full(shape):
    return pl.BlockSpec(shape, lambda i: tuple(0 for _ in shape))


def _rows(blk, width):
    return pl.BlockSpec((blk, width), lambda i: (i, 0))


def _stage_d(agg, xji, x1, rbf0, W_up, Wb0_1, bb0_1, Wb0_2, bb0_2,
             W_lin, b_lin, Wa0_1, ba0_1, Wa0_2, ba0_2,
             Wa1_1, ba1_1, Wa1_2, ba1_2, W_rbf):
    grid = (_E // _BLK_D,)
    return pl.pallas_call(
        _stage_d_body,
        grid=grid,
        in_specs=[
            _rows(_BLK_D, _IE),
            _rows(_BLK_D, _H), _rows(_BLK_D, _H), _rows(_BLK_D, _NR),
            _full((_IE, _H)),
            _full((_H, _H)), _full((1, _H)), _full((_H, _H)), _full((1, _H)),
            _full((_H, _H)), _full((1, _H)),
            _full((_H, _H)), _full((1, _H)), _full((_H, _H)), _full((1, _H)),
            _full((_H, _H)), _full((1, _H)), _full((_H, _H)), _full((1, _H)),
            _full((_NR, _H)),
        ],
        out_specs=[_rows(_BLK_D, _H), _rows(_BLK_D, _H)],
        out_shape=[
            jax.ShapeDtypeStruct((_E, _H), jnp.float32),
            jax.ShapeDtypeStruct((_E, _H), jnp.float32),
        ],
        compiler_params=pltpu.CompilerParams(
            dimension_semantics=("parallel",)),
    )(agg, xji, x1, rbf0, W_up, Wb0_1, bb0_1, Wb0_2, bb0_2, W_lin, b_lin,
      Wa0_1, ba0_1, Wa0_2, ba0_2, Wa1_1, ba1_1, Wa1_2, ba1_2, W_rbf)


def kernel(x1, x2, rbf0, sbf, W_rbf1, W_rbf2, W_sbf1, W_sbf2, W_rbf, W_kj,
           b_kj, W_ji, b_ji, W_down, W_up, Wb0_1, bb0_1, Wb0_2, bb0_2,
           W_lin, b_lin, Wa0_1, ba0_1, Wa0_2, ba0_2, Wa1_1, ba1_1, Wa1_2,
           ba1_2, idx_kj, idx_ji):
    del x2
    r2 = lambda b: b.reshape(1, _H)
    xji, xd = _stage_a(x1, rbf0, W_ji, r2(b_ji), W_kj, r2(b_kj),
                       W_rbf1, W_rbf2, W_down)
    sbf2 = _stage_b(sbf, W_sbf1, W_sbf2)
    kj2 = idx_kj.astype(jnp.int32).reshape(_R, _GV)
    ji2 = idx_ji.astype(jnp.int32).reshape(_R, _GV)
    zeros = jnp.zeros((_C + _G, _NF), jnp.float32)
    xdT = xd.reshape(_E, _NSL, _NF).transpose(1, 0, 2).reshape(_NSL * _E, _NF)
    agg = _sc_sparse(xdT, sbf2, kj2, ji2, zeros)
    e1, e2 = _stage_d(agg, xji, x1, rbf0, W_up, Wb0_1, r2(bb0_1), Wb0_2,
                      r2(bb0_2), W_lin, r2(b_lin), Wa0_1, r2(ba0_1), Wa0_2,
                      r2(ba0_2), Wa1_1, r2(ba1_1), Wa1_2, r2(ba1_2), W_rbf)
    return (e1, e2)


# speedup vs baseline: 4.8774x; 1.1227x over previous
import functools

import jax
import jax.numpy as jnp
from jax import lax
from jax.experimental import pallas as pl
from jax.experimental.pallas import tpu as pltpu
from jax.experimental.pallas import tpu_sc as plsc

_E = 160000
_T = 640000
_H = 256
_NR = 6
_NSR = 42
_IE = 64

_BLK_A = 2000
_BLK_B = 8000
_BLK_D = 2000

_NF = 16
_NSL = 4
_GV = 64
_K = 5
_G = 2048
_NSUB = 16
_UNROLL = 8

_R = _T // _GV
_CPT = _R // _NSUB
_NCH = _CPT // _K
_C = _E // 2
_ZPT = (_C + _G) // _NSUB
_DPT = _C // _NSUB


def _swish(x):
    return x * jax.nn.sigmoid(x)


def _mm(a, b):
    return jax.lax.dot_general(a, b, (((1,), (0,)), ((), ())),
                               preferred_element_type=jnp.float32)


def _stage_a_body(x1_ref, rbf0_ref, Wji_ref, bji_ref, Wkj_ref, bkj_ref,
                  Wr1_ref, Wr2_ref, Wdown_ref, xji_ref, xd_ref):
    x1 = x1_ref[...]
    xji_ref[...] = _swish(_mm(x1, Wji_ref[...]) + bji_ref[...])
    rbf = _mm(_mm(rbf0_ref[...], Wr1_ref[...]), Wr2_ref[...])
    xkj = _swish(_mm(x1, Wkj_ref[...]) + bkj_ref[...]) * rbf
    xd_ref[...] = _swish(_mm(xkj, Wdown_ref[...]))


def _stage_a(x1, rbf0, W_ji, b_ji, W_kj, b_kj, W_rbf1, W_rbf2, W_down):
    grid = (_E // _BLK_A,)
    return pl.pallas_call(
        _stage_a_body,
        grid=grid,
        in_specs=[
            _rows(_BLK_A, _H), _rows(_BLK_A, _NR),
            _full((_H, _H)), _full((1, _H)), _full((_H, _H)), _full((1, _H)),
            _full((_NR, 8)), _full((8, _H)), _full((_H, _IE)),
        ],
        out_specs=[_rows(_BLK_A, _H), _rows(_BLK_A, _IE)],
        out_shape=[jax.ShapeDtypeStruct((_E, _H), jnp.float32),
                   jax.ShapeDtypeStruct((_E, _IE), jnp.float32)],
        compiler_params=pltpu.CompilerParams(
            dimension_semantics=("parallel",)),
    )(x1, rbf0, W_ji, b_ji, W_kj, b_kj, W_rbf1, W_rbf2, W_down)


def _stage_b_body(sbf_ref, Ws1_ref, Ws2_ref, sbf2_ref):
    sbf2_ref[...] = _mm(_mm(sbf_ref[...], Ws1_ref[...]), Ws2_ref[...])


def _stage_b(sbf, W_sbf1, W_sbf2):
    grid = (_T // _BLK_B,)
    return pl.pallas_call(
        _stage_b_body,
        grid=grid,
        in_specs=[_rows(_BLK_B, _NSR), _full((_NSR, 8)), _full((8, _IE))],
        out_specs=_rows(_BLK_B, _IE),
        out_shape=jax.ShapeDtypeStruct((_T, _IE), jnp.float32),
        compiler_params=pltpu.CompilerParams(
            dimension_semantics=("parallel",)),
    )(sbf, W_sbf1, W_sbf2)


_sc_mesh = plsc.VectorSubcoreMesh(core_axis_name="c", subcore_axis_name="s")


@pl.kernel(
    out_type=jax.ShapeDtypeStruct((_E, _IE), jnp.float32),
    mesh=_sc_mesh,
    scratch_types=[
        pltpu.VMEM((_K, _GV), jnp.int32),
        pltpu.VMEM((_K, _GV), jnp.int32),
        pltpu.VMEM((_K * _GV, _NF), jnp.float32),
        pltpu.VMEM((_K * _GV, _NF), jnp.float32),
        pltpu.VMEM_SHARED((_C + _G, _NF), jnp.float32),
        pltpu.SemaphoreType.DMA,
        pltpu.SemaphoreType.DMA,
        pltpu.SemaphoreType.DMA,
    ],
    compiler_params=pltpu.CompilerParams(use_tc_tiling_on_sc=False),
)
def _sc_sparse(xdT, sbf2, kj4, dest2, zeros_hbm, out,
               kj_b, dest_b, rows_b, sb_b, acc, isem, gsem, ssem):
    c = lax.axis_index("c")
    w = lax.axis_index("s")
    row0 = w * _CPT

    for r in range(_NSL):
        f0 = r * _NF
        pltpu.sync_copy(zeros_hbm.at[pl.ds(w * _ZPT, _ZPT)],
                        acc.at[pl.ds(w * _ZPT, _ZPT)])
        plsc.subcore_barrier()

        def chunk_body(ci, _, f0=f0, r=r):
            r0 = row0 + ci * _K
            kj_cp = pltpu.async_copy(kj4.at[r].at[pl.ds(r0, _K)], kj_b, isem)
            d_cp = pltpu.async_copy(dest2.at[c].at[pl.ds(r0, _K)],
                                    dest_b, isem)
            sb_cp = pltpu.async_copy(
                sbf2.at[pl.ds(r0 * _GV, _K * _GV), pl.ds(f0, _NF)],
                sb_b, gsem)
            kj_cp.wait()
            gathers = [
                pltpu.async_copy(xdT.at[kj_b.at[kk]],
                                 rows_b.at[pl.ds(kk * _GV, _GV)], gsem)
                for kk in range(_K)
            ]
            sb_cp.wait()
            for g in gathers:
                g.wait()

            def mul_body(i, _):
                for u in range(_UNROLL):
                    rr = i * _UNROLL + u
                    rows_b[rr, :] = rows_b[rr, :] * sb_b[rr, :]
                return 0

            lax.fori_loop(0, _K * _GV // _UNROLL, mul_body, 0)
            d_cp.wait()
            scatters = [
                pltpu.async_copy(rows_b.at[pl.ds(kk * _GV, _GV)],
                                 acc.at[dest_b.at[kk]], ssem, add=True)
                for kk in range(_K)
            ]
            for s in scatters:
                s.wait()
            return 0

        lax.fori_loop(0, _NCH, chunk_body, 0)
        plsc.subcore_barrier()
        pltpu.sync_copy(acc.at[pl.ds(w * _DPT, _DPT)],
                        out.at[pl.ds(c * _C + w * _DPT, _DPT),
                               pl.ds(f0, _NF)])
        plsc.subcore_barrier()


def _stage_d_body(agg_ref, xji_ref, x1_ref, rbf0_ref,
                  Wup_ref,
                  Wb01_ref, bb01_ref, Wb02_ref, bb02_ref,
                  Wlin_ref, blin_ref,
                  Wa01_ref, ba01_ref, Wa02_ref, ba02_ref,
                  Wa11_ref, ba11_ref, Wa12_ref, ba12_ref,
                  Wrbf_ref, e1_ref, e2_ref):
    xu = _swish(_mm(agg_ref[...], Wup_ref[...]))
    e1 = xji_ref[...] + xu
    e1 = e1 + _swish(_mm(_swish(_mm(e1, Wb01_ref[...]) + bb01_ref[...]),
                         Wb02_ref[...]) + bb02_ref[...])
    e1 = _swish(_mm(e1, Wlin_ref[...]) + blin_ref[...]) + x1_ref[...]
    e1 = e1 + _swish(_mm(_swish(_mm(e1, Wa01_ref[...]) + ba01_ref[...]),
                         Wa02_ref[...]) + ba02_ref[...])
    e1 = e1 + _swish(_mm(_swish(_mm(e1, Wa11_ref[...]) + ba11_ref[...]),
                         Wa12_ref[...]) + ba12_ref[...])
    e1_ref[...] = e1
    e2_ref[...] = _mm(rbf0_ref[...], Wrbf_ref[...]) * e1


def _full(shape):
    return pl.BlockSpec(shape, lambda i: tuple(0 for _ in shape))


def _rows(blk, width):
    return pl.BlockSpec((blk, width), lambda i: (i, 0))


def _stage_d(agg, xji, x1, rbf0, W_up, Wb0_1, bb0_1, Wb0_2, bb0_2,
             W_lin, b_lin, Wa0_1, ba0_1, Wa0_2, ba0_2,
             Wa1_1, ba1_1, Wa1_2, ba1_2, W_rbf):
    grid = (_E // _BLK_D,)
    return pl.pallas_call(
        _stage_d_body,
        grid=grid,
        in_specs=[
            _rows(_BLK_D, _IE),
            _rows(_BLK_D, _H), _rows(_BLK_D, _H), _rows(_BLK_D, _NR),
            _full((_IE, _H)),
            _full((_H, _H)), _full((1, _H)), _full((_H, _H)), _full((1, _H)),
            _full((_H, _H)), _full((1, _H)),
            _full((_H, _H)), _full((1, _H)), _full((_H, _H)), _full((1, _H)),
            _full((_H, _H)), _full((1, _H)), _full((_H, _H)), _full((1, _H)),
            _full((_NR, _H)),
        ],
        out_specs=[_rows(_BLK_D, _H), _rows(_BLK_D, _H)],
        out_shape=[
            jax.ShapeDtypeStruct((_E, _H), jnp.float32),
            jax.ShapeDtypeStruct((_E, _H), jnp.float32),
        ],
        compiler_params=pltpu.CompilerParams(
            dimension_semantics=("parallel",)),
    )(agg, xji, x1, rbf0, W_up, Wb0_1, bb0_1, Wb0_2, bb0_2, W_lin, b_lin,
      Wa0_1, ba0_1, Wa0_2, ba0_2, Wa1_1, ba1_1, Wa1_2, ba1_2, W_rbf)


def kernel(x1, x2, rbf0, sbf, W_rbf1, W_rbf2, W_sbf1, W_sbf2, W_rbf, W_kj,
           b_kj, W_ji, b_ji, W_down, W_up, Wb0_1, bb0_1, Wb0_2, bb0_2,
           W_lin, b_lin, Wa0_1, ba0_1, Wa0_2, ba0_2, Wa1_1, ba1_1, Wa1_2,
           ba1_2, idx_kj, idx_ji):
    del x2
    r2 = lambda b: b.reshape(1, _H)
    xji, xd = _stage_a(x1, rbf0, W_ji, r2(b_ji), W_kj, r2(b_kj),
                       W_rbf1, W_rbf2, W_down)
    sbf2 = _stage_b(sbf, W_sbf1, W_sbf2)
    kj2 = idx_kj.astype(jnp.int32).reshape(_R, _GV)
    ji2 = idx_ji.astype(jnp.int32).reshape(_R, _GV)
    kj4 = kj2[None, :, :] + (jnp.arange(_NSL, dtype=jnp.int32)
                             * _E)[:, None, None]
    d0 = jnp.where((ji2 >= 0) & (ji2 < _C), ji2, (ji2 & (_G - 1)) + _C)
    j1 = ji2 - _C
    d1 = jnp.where((j1 >= 0) & (j1 < _C), j1, (ji2 & (_G - 1)) + _C)
    dest2 = jnp.stack([d0, d1])
    zeros = jnp.zeros((_C + _G, _NF), jnp.float32)
    xdT = xd.reshape(_E, _NSL, _NF).transpose(1, 0, 2).reshape(_NSL * _E, _NF)
    agg = _sc_sparse(xdT, sbf2, kj4, dest2, zeros)
    e1, e2 = _stage_d(agg, xji, x1, rbf0, W_up, Wb0_1, r2(bb0_1), Wb0_2,
                      r2(bb0_2), W_lin, r2(b_lin), Wa0_1, r2(ba0_1), Wa0_2,
                      r2(ba0_2), Wa1_1, r2(ba1_1), Wa1_2, r2(ba1_2), W_rbf)
    return (e1, e2)


# speedup vs baseline: 5.3496x; 1.0968x over previous
import functools

import jax
import jax.numpy as jnp
from jax import lax
from jax.experimental import pallas as pl
from jax.experimental.pallas import tpu as pltpu
from jax.experimental.pallas import tpu_sc as plsc

_E = 160000
_T = 640000
_H = 256
_NR = 6
_NSR = 42
_IE = 64

_BLK_A = 2000
_BLK_B = 8000
_BLK_D = 2000

_NF = 16
_NSL = 4
_GV = 80
_K = 10
_G = 2048
_NSUB = 16
_UNROLL = 8

_R = _T // _GV
_CPT = _R // _NSUB
_NCH = _CPT // _K
_C = _E // 2
_ZPT = (_C + _G) // _NSUB
_DPT = _C // _NSUB


def _swish(x):
    return x * jax.nn.sigmoid(x)


def _mm(a, b):
    return jax.lax.dot_general(a, b, (((1,), (0,)), ((), ())),
                               preferred_element_type=jnp.float32)


def _stage_a_body(x1_ref, rbf0_ref, Wji_ref, bji_ref, Wkj_ref, bkj_ref,
                  Wr1_ref, Wr2_ref, Wdown_ref, xji_ref, xd_ref):
    x1 = x1_ref[...]
    xji_ref[...] = _swish(_mm(x1, Wji_ref[...]) + bji_ref[...])
    rbf = _mm(_mm(rbf0_ref[...], Wr1_ref[...]), Wr2_ref[...])
    xkj = _swish(_mm(x1, Wkj_ref[...]) + bkj_ref[...]) * rbf
    xd_ref[...] = _swish(_mm(xkj, Wdown_ref[...]))


def _stage_a(x1, rbf0, W_ji, b_ji, W_kj, b_kj, W_rbf1, W_rbf2, W_down):
    grid = (_E // _BLK_A,)
    return pl.pallas_call(
        _stage_a_body,
        grid=grid,
        in_specs=[
            _rows(_BLK_A, _H), _rows(_BLK_A, _NR),
            _full((_H, _H)), _full((1, _H)), _full((_H, _H)), _full((1, _H)),
            _full((_NR, 8)), _full((8, _H)), _full((_H, _IE)),
        ],
        out_specs=[_rows(_BLK_A, _H), _rows(_BLK_A, _IE)],
        out_shape=[jax.ShapeDtypeStruct((_E, _H), jnp.float32),
                   jax.ShapeDtypeStruct((_E, _IE), jnp.float32)],
        compiler_params=pltpu.CompilerParams(
            dimension_semantics=("parallel",)),
    )(x1, rbf0, W_ji, b_ji, W_kj, b_kj, W_rbf1, W_rbf2, W_down)


def _stage_b_body(sbf_ref, Ws1_ref, Ws2_ref, sbf2_ref):
    sbf2_ref[...] = _mm(_mm(sbf_ref[...], Ws1_ref[...]), Ws2_ref[...])


def _stage_b(sbf, W_sbf1, W_sbf2):
    grid = (_T // _BLK_B,)
    return pl.pallas_call(
        _stage_b_body,
        grid=grid,
        in_specs=[_rows(_BLK_B, _NSR), _full((_NSR, 8)), _full((8, _IE))],
        out_specs=_rows(_BLK_B, _IE),
        out_shape=jax.ShapeDtypeStruct((_T, _IE), jnp.float32),
        compiler_params=pltpu.CompilerParams(
            dimension_semantics=("parallel",)),
    )(sbf, W_sbf1, W_sbf2)


_sc_mesh = plsc.VectorSubcoreMesh(core_axis_name="c", subcore_axis_name="s")


@pl.kernel(
    out_type=jax.ShapeDtypeStruct((_E, _IE), jnp.float32),
    mesh=_sc_mesh,
    scratch_types=[
        pltpu.VMEM((_K, _GV), jnp.int32),
        pltpu.VMEM((_K, _GV), jnp.int32),
        pltpu.VMEM((_K * _GV, _NF), jnp.float32),
        pltpu.VMEM((_K * _GV, _NF), jnp.float32),
        pltpu.VMEM_SHARED((_C + _G, _NF), jnp.float32),
        pltpu.SemaphoreType.DMA,
        pltpu.SemaphoreType.DMA,
        pltpu.SemaphoreType.DMA,
    ],
    compiler_params=pltpu.CompilerParams(use_tc_tiling_on_sc=False),
)
def _sc_sparse(xdT, sbf2, kj4, dest2, zeros_hbm, out,
               kj_b, dest_b, rows_b, sb_b, acc, isem, gsem, ssem):
    c = lax.axis_index("c")
    w = lax.axis_index("s")
    row0 = w * _CPT

    for r in range(_NSL):
        f0 = r * _NF
        pltpu.sync_copy(zeros_hbm.at[pl.ds(w * _ZPT, _ZPT)],
                        acc.at[pl.ds(w * _ZPT, _ZPT)])
        plsc.subcore_barrier()

        def chunk_body(ci, _, f0=f0, r=r):
            r0 = row0 + ci * _K
            kj_cp = pltpu.async_copy(kj4.at[r].at[pl.ds(r0, _K)], kj_b, isem)
            d_cp = pltpu.async_copy(dest2.at[c].at[pl.ds(r0, _K)],
                                    dest_b, isem)
            sb_cp = pltpu.async_copy(
                sbf2.at[pl.ds(r0 * _GV, _K * _GV), pl.ds(f0, _NF)],
                sb_b, gsem)
            kj_cp.wait()
            gathers = [
                pltpu.async_copy(xdT.at[kj_b.at[kk]],
                                 rows_b.at[pl.ds(kk * _GV, _GV)], gsem)
                for kk in range(_K)
            ]
            sb_cp.wait()
            for g in gathers:
                g.wait()

            def mul_body(i, _):
                for u in range(_UNROLL):
                    rr = i * _UNROLL + u
                    rows_b[rr, :] = rows_b[rr, :] * sb_b[rr, :]
                return 0

            lax.fori_loop(0, _K * _GV // _UNROLL, mul_body, 0)
            d_cp.wait()
            scatters = [
                pltpu.async_copy(rows_b.at[pl.ds(kk * _GV, _GV)],
                                 acc.at[dest_b.at[kk]], ssem, add=True)
                for kk in range(_K)
            ]
            for s in scatters:
                s.wait()
            return 0

        lax.fori_loop(0, _NCH, chunk_body, 0)
        plsc.subcore_barrier()
        pltpu.sync_copy(acc.at[pl.ds(w * _DPT, _DPT)],
                        out.at[pl.ds(c * _C + w * _DPT, _DPT),
                               pl.ds(f0, _NF)])
        plsc.subcore_barrier()


def _stage_d_body(agg_ref, xji_ref, x1_ref, rbf0_ref,
                  Wup_ref,
                  Wb01_ref, bb01_ref, Wb02_ref, bb02_ref,
                  Wlin_ref, blin_ref,
                  Wa01_ref, ba01_ref, Wa02_ref, ba02_ref,
                  Wa11_ref, ba11_ref, Wa12_ref, ba12_ref,
                  Wrbf_ref, e1_ref, e2_ref):
    xu = _swish(_mm(agg_ref[...], Wup_ref[...]))
    e1 = xji_ref[...] + xu
    e1 = e1 + _swish(_mm(_swish(_mm(e1, Wb01_ref[...]) + bb01_ref[...]),
                         Wb02_ref[...]) + bb02_ref[...])
    e1 = _swish(_mm(e1, Wlin_ref[...]) + blin_ref[...]) + x1_ref[...]
    e1 = e1 + _swish(_mm(_swish(_mm(e1, Wa01_ref[...]) + ba01_ref[...]),
                         Wa02_ref[...]) + ba02_ref[...])
    e1 = e1 + _swish(_mm(_swish(_mm(e1, Wa11_ref[...]) + ba11_ref[...]),
                         Wa12_ref[...]) + ba12_ref[...])
    e1_ref[...] = e1
    e2_ref[...] = _mm(rbf0_ref[...], Wrbf_ref[...]) * e1


def _full(shape):
    return pl.BlockSpec(shape, lambda i: tuple(0 for _ in shape))


def _rows(blk, width):
    return pl.BlockSpec((blk, width), lambda i: (i, 0))


def _stage_d(agg, xji, x1, rbf0, W_up, Wb0_1, bb0_1, Wb0_2, bb0_2,
             W_lin, b_lin, Wa0_1, ba0_1, Wa0_2, ba0_2,
             Wa1_1, ba1_1, Wa1_2, ba1_2, W_rbf):
    grid = (_E // _BLK_D,)
    return pl.pallas_call(
        _stage_d_body,
        grid=grid,
        in_specs=[
            _rows(_BLK_D, _IE),
            _rows(_BLK_D, _H), _rows(_BLK_D, _H), _rows(_BLK_D, _NR),
            _full((_IE, _H)),
            _full((_H, _H)), _full((1, _H)), _full((_H, _H)), _full((1, _H)),
            _full((_H, _H)), _full((1, _H)),
            _full((_H, _H)), _full((1, _H)), _full((_H, _H)), _full((1, _H)),
            _full((_H, _H)), _full((1, _H)), _full((_H, _H)), _full((1, _H)),
            _full((_NR, _H)),
        ],
        out_specs=[_rows(_BLK_D, _H), _rows(_BLK_D, _H)],
        out_shape=[
            jax.ShapeDtypeStruct((_E, _H), jnp.float32),
            jax.ShapeDtypeStruct((_E, _H), jnp.float32),
        ],
        compiler_params=pltpu.CompilerParams(
            dimension_semantics=("parallel",)),
    )(agg, xji, x1, rbf0, W_up, Wb0_1, bb0_1, Wb0_2, bb0_2, W_lin, b_lin,
      Wa0_1, ba0_1, Wa0_2, ba0_2, Wa1_1, ba1_1, Wa1_2, ba1_2, W_rbf)


def kernel(x1, x2, rbf0, sbf, W_rbf1, W_rbf2, W_sbf1, W_sbf2, W_rbf, W_kj,
           b_kj, W_ji, b_ji, W_down, W_up, Wb0_1, bb0_1, Wb0_2, bb0_2,
           W_lin, b_lin, Wa0_1, ba0_1, Wa0_2, ba0_2, Wa1_1, ba1_1, Wa1_2,
           ba1_2, idx_kj, idx_ji):
    del x2
    r2 = lambda b: b.reshape(1, _H)
    xji, xd = _stage_a(x1, rbf0, W_ji, r2(b_ji), W_kj, r2(b_kj),
                       W_rbf1, W_rbf2, W_down)
    sbf2 = _stage_b(sbf, W_sbf1, W_sbf2)
    kj2 = idx_kj.astype(jnp.int32).reshape(_R, _GV)
    ji2 = idx_ji.astype(jnp.int32).reshape(_R, _GV)
    kj4 = kj2[None, :, :] + (jnp.arange(_NSL, dtype=jnp.int32)
                             * _E)[:, None, None]
    d0 = jnp.where((ji2 >= 0) & (ji2 < _C), ji2, (ji2 & (_G - 1)) + _C)
    j1 = ji2 - _C
    d1 = jnp.where((j1 >= 0) & (j1 < _C), j1, (ji2 & (_G - 1)) + _C)
    dest2 = jnp.stack([d0, d1])
    zeros = jnp.zeros((_C + _G, _NF), jnp.float32)
    xdT = xd.reshape(_E, _NSL, _NF).transpose(1, 0, 2).reshape(_NSL * _E, _NF)
    agg = _sc_sparse(xdT, sbf2, kj4, dest2, zeros)
    e1, e2 = _stage_d(agg, xji, x1, rbf0, W_up, Wb0_1, r2(bb0_1), Wb0_2,
                      r2(bb0_2), W_lin, r2(b_lin), Wa0_1, r2(ba0_1), Wa0_2,
                      r2(ba0_2), Wa1_1, r2(ba1_1), Wa1_2, r2(ba1_2), W_rbf)
    return (e1, e2)


# speedup vs baseline: 5.5690x; 1.0410x over previous
import functools

import jax
import jax.numpy as jnp
from jax import lax
from jax.experimental import pallas as pl
from jax.experimental.pallas import tpu as pltpu
from jax.experimental.pallas import tpu_sc as plsc

_E = 160000
_T = 640000
_H = 256
_NR = 6
_NSR = 42
_IE = 64

_BLK_A = 2000
_BLK_B = 8000
_BLK_D = 2000

_NF = 16
_NSL = 4
_GV = 80
_K = 10
_G = 2048
_NSUB = 16
_UNROLL = 8

_R = _T // _GV
_CPT = _R // _NSUB
_NCH = _CPT // _K
_C = _E // 2
_ZPT = (_C + _G) // _NSUB
_DPT = _C // _NSUB


def _swish(x):
    return x * jax.nn.sigmoid(x)


def _mm(a, b):
    return jax.lax.dot_general(a, b, (((1,), (0,)), ((), ())),
                               preferred_element_type=jnp.float32)


def _stage_a_body(x1_ref, rbf0_ref, Wji_ref, bji_ref, Wkj_ref, bkj_ref,
                  Wr1_ref, Wr2_ref, Wdown_ref, xji_ref, xd_ref):
    x1 = x1_ref[...]
    xji_ref[...] = _swish(_mm(x1, Wji_ref[...]) + bji_ref[...])
    rbf = _mm(_mm(rbf0_ref[...], Wr1_ref[...]), Wr2_ref[...])
    xkj = _swish(_mm(x1, Wkj_ref[...]) + bkj_ref[...]) * rbf
    xd_ref[...] = _swish(_mm(xkj, Wdown_ref[...]))


def _stage_a(x1, rbf0, W_ji, b_ji, W_kj, b_kj, W_rbf1, W_rbf2, W_down):
    grid = (_E // _BLK_A,)
    return pl.pallas_call(
        _stage_a_body,
        grid=grid,
        in_specs=[
            _rows(_BLK_A, _H), _rows(_BLK_A, _NR),
            _full((_H, _H)), _full((1, _H)), _full((_H, _H)), _full((1, _H)),
            _full((_NR, 8)), _full((8, _H)), _full((_H, _IE)),
        ],
        out_specs=[_rows(_BLK_A, _H), _rows(_BLK_A, _IE)],
        out_shape=[jax.ShapeDtypeStruct((_E, _H), jnp.float32),
                   jax.ShapeDtypeStruct((_E, _IE), jnp.float32)],
        compiler_params=pltpu.CompilerParams(
            dimension_semantics=("parallel",)),
    )(x1, rbf0, W_ji, b_ji, W_kj, b_kj, W_rbf1, W_rbf2, W_down)


def _stage_b_body(sbf_ref, Ws1_ref, Ws2_ref, sbf2_ref):
    sbf2_ref[...] = _mm(_mm(sbf_ref[...], Ws1_ref[...]), Ws2_ref[...])


def _stage_b(sbf, W_sbf1, W_sbf2):
    grid = (_T // _BLK_B,)
    return pl.pallas_call(
        _stage_b_body,
        grid=grid,
        in_specs=[_rows(_BLK_B, _NSR), _full((_NSR, 8)), _full((8, _IE))],
        out_specs=_rows(_BLK_B, _IE),
        out_shape=jax.ShapeDtypeStruct((_T, _IE), jnp.float32),
        compiler_params=pltpu.CompilerParams(
            dimension_semantics=("parallel",)),
    )(sbf, W_sbf1, W_sbf2)


_sc_mesh = plsc.VectorSubcoreMesh(core_axis_name="c", subcore_axis_name="s")


@pl.kernel(
    out_type=jax.ShapeDtypeStruct((_E, _IE), jnp.float32),
    mesh=_sc_mesh,
    scratch_types=[
        pltpu.VMEM((_K, _GV), jnp.int32),
        pltpu.VMEM((_K, _GV), jnp.int32),
        pltpu.VMEM((_K, _GV), jnp.int32),
        pltpu.VMEM((_K * _GV, _NF), jnp.float32),
        pltpu.VMEM((_K * _GV, _NF), jnp.float32),
        pltpu.VMEM((_K * _GV, _NF), jnp.float32),
        pltpu.VMEM_SHARED((_C + _G, _NF), jnp.float32),
        pltpu.SemaphoreType.DMA,
        pltpu.SemaphoreType.DMA,
        pltpu.SemaphoreType.DMA,
        pltpu.SemaphoreType.DMA,
    ],
    compiler_params=pltpu.CompilerParams(use_tc_tiling_on_sc=False),
)
def _sc_sparse(xdT, sbf2, kj4, dest2, zeros_hbm, out,
               kj_b, dest_ba, dest_bb, rows_ba, rows_bb, sb_b, acc,
               isem, gsem, ssema, ssemb):
    c = lax.axis_index("c")
    w = lax.axis_index("s")
    row0 = w * _CPT

    for r in range(_NSL):
        f0 = r * _NF
        pltpu.sync_copy(zeros_hbm.at[pl.ds(w * _ZPT, _ZPT)],
                        acc.at[pl.ds(w * _ZPT, _ZPT)])
        plsc.subcore_barrier()

        def process(ci, rows_b, dest_b, ssem, guard, f0=f0, r=r):
            r0 = row0 + ci * _K
            kj_cp = pltpu.async_copy(kj4.at[r].at[pl.ds(r0, _K)], kj_b, isem)
            d_cp = pltpu.async_copy(dest2.at[c].at[pl.ds(r0, _K)],
                                    dest_b, isem)
            sb_cp = pltpu.async_copy(
                sbf2.at[pl.ds(r0 * _GV, _K * _GV), pl.ds(f0, _NF)],
                sb_b, gsem)
            kj_cp.wait()

            @pl.when(guard)
            def _():
                pltpu.make_async_copy(
                    zeros_hbm.at[pl.ds(0, _K * _GV)], rows_b, ssem).wait()

            gathers = [
                pltpu.async_copy(xdT.at[kj_b.at[kk]],
                                 rows_b.at[pl.ds(kk * _GV, _GV)], gsem)
                for kk in range(_K)
            ]
            sb_cp.wait()
            for g in gathers:
                g.wait()

            def mul_body(i, _):
                for u in range(_UNROLL):
                    rr = i * _UNROLL + u
                    rows_b[rr, :] = rows_b[rr, :] * sb_b[rr, :]
                return 0

            lax.fori_loop(0, _K * _GV // _UNROLL, mul_body, 0)
            d_cp.wait()
            for kk in range(_K):
                pltpu.async_copy(rows_b.at[pl.ds(kk * _GV, _GV)],
                                 acc.at[dest_b.at[kk]], ssem, add=True)

        def pair_body(i, _):
            process(2 * i, rows_ba, dest_ba, ssema, i > 0)
            process(2 * i + 1, rows_bb, dest_bb, ssemb, i > 0)
            return 0

        lax.fori_loop(0, _NCH // 2, pair_body, 0)
        pltpu.make_async_copy(zeros_hbm.at[pl.ds(0, _K * _GV)],
                              rows_ba, ssema).wait()
        pltpu.make_async_copy(zeros_hbm.at[pl.ds(0, _K * _GV)],
                              rows_bb, ssemb).wait()
        plsc.subcore_barrier()
        pltpu.sync_copy(acc.at[pl.ds(w * _DPT, _DPT)],
                        out.at[pl.ds(c * _C + w * _DPT, _DPT),
                               pl.ds(f0, _NF)])
        plsc.subcore_barrier()


def _stage_d_body(agg_ref, xji_ref, x1_ref, rbf0_ref,
                  Wup_ref,
                  Wb01_ref, bb01_ref, Wb02_ref, bb02_ref,
                  Wlin_ref, blin_ref,
                  Wa01_ref, ba01_ref, Wa02_ref, ba02_ref,
                  Wa11_ref, ba11_ref, Wa12_ref, ba12_ref,
                  Wrbf_ref, e1_ref, e2_ref):
    xu = _swish(_mm(agg_ref[...], Wup_ref[...]))
    e1 = xji_ref[...] + xu
    e1 = e1 + _swish(_mm(_swish(_mm(e1, Wb01_ref[...]) + bb01_ref[...]),
                         Wb02_ref[...]) + bb02_ref[...])
    e1 = _swish(_mm(e1, Wlin_ref[...]) + blin_ref[...]) + x1_ref[...]
    e1 = e1 + _swish(_mm(_swish(_mm(e1, Wa01_ref[...]) + ba01_ref[...]),
                         Wa02_ref[...]) + ba02_ref[...])
    e1 = e1 + _swish(_mm(_swish(_mm(e1, Wa11_ref[...]) + ba11_ref[...]),
                         Wa12_ref[...]) + ba12_ref[...])
    e1_ref[...] = e1
    e2_ref[...] = _mm(rbf0_ref[...], Wrbf_ref[...]) * e1


def _full(shape):
    return pl.BlockSpec(shape, lambda i: tuple(0 for _ in shape))


def _rows(blk, width):
    return pl.BlockSpec((blk, width), lambda i: (i, 0))


def _stage_d(agg, xji, x1, rbf0, W_up, Wb0_1, bb0_1, Wb0_2, bb0_2,
             W_lin, b_lin, Wa0_1, ba0_1, Wa0_2, ba0_2,
             Wa1_1, ba1_1, Wa1_2, ba1_2, W_rbf):
    grid = (_E // _BLK_D,)
    return pl.pallas_call(
        _stage_d_body,
        grid=grid,
        in_specs=[
            _rows(_BLK_D, _IE),
            _rows(_BLK_D, _H), _rows(_BLK_D, _H), _rows(_BLK_D, _NR),
            _full((_IE, _H)),
            _full((_H, _H)), _full((1, _H)), _full((_H, _H)), _full((1, _H)),
            _full((_H, _H)), _full((1, _H)),
            _full((_H, _H)), _full((1, _H)), _full((_H, _H)), _full((1, _H)),
            _full((_H, _H)), _full((1, _H)), _full((_H, _H)), _full((1, _H)),
            _full((_NR, _H)),
        ],
        out_specs=[_rows(_BLK_D, _H), _rows(_BLK_D, _H)],
        out_shape=[
            jax.ShapeDtypeStruct((_E, _H), jnp.float32),
            jax.ShapeDtypeStruct((_E, _H), jnp.float32),
        ],
        compiler_params=pltpu.CompilerParams(
            dimension_semantics=("parallel",)),
    )(agg, xji, x1, rbf0, W_up, Wb0_1, bb0_1, Wb0_2, bb0_2, W_lin, b_lin,
      Wa0_1, ba0_1, Wa0_2, ba0_2, Wa1_1, ba1_1, Wa1_2, ba1_2, W_rbf)


def kernel(x1, x2, rbf0, sbf, W_rbf1, W_rbf2, W_sbf1, W_sbf2, W_rbf, W_kj,
           b_kj, W_ji, b_ji, W_down, W_up, Wb0_1, bb0_1, Wb0_2, bb0_2,
           W_lin, b_lin, Wa0_1, ba0_1, Wa0_2, ba0_2, Wa1_1, ba1_1, Wa1_2,
           ba1_2, idx_kj, idx_ji):
    del x2
    r2 = lambda b: b.reshape(1, _H)
    xji, xd = _stage_a(x1, rbf0, W_ji, r2(b_ji), W_kj, r2(b_kj),
                       W_rbf1, W_rbf2, W_down)
    sbf2 = _stage_b(sbf, W_sbf1, W_sbf2)
    kj2 = idx_kj.astype(jnp.int32).reshape(_R, _GV)
    ji2 = idx_ji.astype(jnp.int32).reshape(_R, _GV)
    kj4 = kj2[None, :, :] + (jnp.arange(_NSL, dtype=jnp.int32)
                             * _E)[:, None, None]
    d0 = jnp.where((ji2 >= 0) & (ji2 < _C), ji2, (ji2 & (_G - 1)) + _C)
    j1 = ji2 - _C
    d1 = jnp.where((j1 >= 0) & (j1 < _C), j1, (ji2 & (_G - 1)) + _C)
    dest2 = jnp.stack([d0, d1])
    zeros = jnp.zeros((_C + _G, _NF), jnp.float32)
    xdT = xd.reshape(_E, _NSL, _NF).transpose(1, 0, 2).reshape(_NSL * _E, _NF)
    agg = _sc_sparse(xdT, sbf2, kj4, dest2, zeros)
    e1, e2 = _stage_d(agg, xji, x1, rbf0, W_up, Wb0_1, r2(bb0_1), Wb0_2,
                      r2(bb0_2), W_lin, r2(b_lin), Wa0_1, r2(ba0_1), Wa0_2,
                      r2(ba0_2), Wa1_1, r2(ba1_1), Wa1_2, r2(ba1_2), W_rbf)
    return (e1, e2)
